# baseline (speedup 1.0000x reference)
import jax
import jax.numpy as jnp
from jax import lax
from jax.experimental import pallas as pl
from jax.experimental.pallas import tpu as pltpu
from jax.experimental.pallas import tpu_sc as plsc

_DIM = 32
_NB = 4
_BLK = 128
_NC = 2
_NS = 16
_NW = _NC * _NS


def _make_lookup(n_s):
  mesh = plsc.VectorSubcoreMesh(
      core_axis_name="c", subcore_axis_name="s",
      num_cores=_NC, num_subcores=_NS)
  n_tok = n_s * _BLK

  def body(table_hbm, idx_hbm, out_hbm, idx_v, cidx, rows, otp, gsem, ssem):
    jb = lax.axis_index("s") * _NC + lax.axis_index("c")
    pltpu.sync_copy(idx_hbm.at[pl.ds(jb * n_tok, n_tok)], idx_v)

    iota = lax.iota(jnp.int32, 16)
    iota_s = iota * n_s

    def extract(s, buf):
      for g in range(_BLK // 16):
        pv = iota_s + (g * 16 * n_s + s)
        cidx[buf, pl.ds(g * 16, 16)] = plsc.load_gather(idx_v, [pv])

    def fire_gather(buf):
      pltpu.async_copy(table_hbm.at[cidx.at[buf]], rows.at[buf],
                       gsem.at[buf])

    def wait_gather(buf):
      pltpu.make_async_copy(table_hbm.at[pl.ds(0, _BLK), :], rows.at[buf],
                            gsem.at[buf]).wait()

    def transpose(rb, ob):
      r = rows.at[rb]
      d_vecs = [jnp.full((16,), d, jnp.int32) for d in range(_DIM)]

      @plsc.parallel_loop(0, _BLK, 16, unroll=2)
      def _(tl0):
        tl_vec = iota + tl0
        for d in range(_DIM):
          v = plsc.load_gather(r, [tl_vec, d_vecs[d]])
          otp[ob, d >> 3, d & 7, pl.ds(tl0, 16)] = v

    def fire_store(s, buf):
      pltpu.async_copy(otp.at[buf], out_hbm.at[s, :, jb], ssem.at[buf])

    def wait_store(s, buf):
      pltpu.make_async_copy(otp.at[buf], out_hbm.at[s, :, jb],
                            ssem.at[buf]).wait()

    for q in range(_NB - 1):
      extract(q, q)
      fire_gather(q)

    def step(i, carry):
      for q in range(_NB):
        s = _NB * i + q

        @pl.when(s + _NB - 1 < n_s)
        def _():
          extract(s + _NB - 1, (q + _NB - 1) % _NB)
          fire_gather((q + _NB - 1) % _NB)

        wait_gather(q)

        @pl.when(s >= 2)
        def _():
          wait_store(s - 2, q & 1)

        transpose(q, q & 1)
        fire_store(s, q & 1)
      return carry

    lax.fori_loop(0, n_s // _NB, step, 0)
    wait_store(n_s - 2, 0)
    wait_store(n_s - 1, 1)

  return pl.kernel(
      body,
      out_type=jax.ShapeDtypeStruct((n_s, _DIM // 8, _NW, 8, _BLK),
                                    jnp.float32),
      mesh=mesh,
      scratch_types=[
          pltpu.VMEM((n_tok,), jnp.int32),
          pltpu.VMEM((_NB, _BLK), jnp.int32),
          pltpu.VMEM((_NB, _BLK, _DIM), jnp.float32),
          pltpu.VMEM((2, _DIM // 8, 8, _BLK), jnp.float32),
          pltpu.SemaphoreType.DMA((_NB,)),
          pltpu.SemaphoreType.DMA((2,)),
      ],
      compiler_params=pltpu.CompilerParams(use_tc_tiling_on_sc=False,
                                           needs_layout_passes=False),
  )


def kernel(x, table):
  r, s = x.shape
  idx = x.reshape(-1).astype(jnp.int32)
  out5 = _make_lookup(s)(table, idx)
  return out5.transpose(2, 4, 0, 1, 3).reshape(r, s, _DIM)


# speedup vs baseline: 1.0047x; 1.0047x over previous
import jax
import jax.numpy as jnp
from jax import lax
from jax.experimental import pallas as pl
from jax.experimental.pallas import tpu as pltpu
from jax.experimental.pallas import tpu_sc as plsc

_DIM = 32
_NB = 4
_BLK = 128
_NC = 2
_NS = 16
_NW = _NC * _NS


def _make_lookup(n_s):
  mesh = plsc.VectorSubcoreMesh(
      core_axis_name="c", subcore_axis_name="s",
      num_cores=_NC, num_subcores=_NS)
  n_tok = n_s * _BLK

  def body(table_hbm, idx_hbm, out_hbm, idx_v, cidx, rows, otp, gsem, ssem):
    jb = lax.axis_index("s") * _NC + lax.axis_index("c")
    pltpu.sync_copy(idx_hbm.at[pl.ds(jb * n_tok, n_tok)], idx_v)

    iota = lax.iota(jnp.int32, 16)
    iota_s = iota * n_s

    def extract(s, buf):
      for g in range(_BLK // 16):
        pv = iota_s + (g * 16 * n_s + s)
        cidx[buf, pl.ds(g * 16, 16)] = plsc.load_gather(idx_v, [pv])

    def fire_gather(buf):
      pltpu.async_copy(table_hbm.at[cidx.at[buf]], rows.at[buf],
                       gsem.at[buf])

    def wait_gather(buf):
      pltpu.make_async_copy(table_hbm.at[pl.ds(0, _BLK), :], rows.at[buf],
                            gsem.at[buf]).wait()

    def transpose(rb, ob):
      r = rows.at[rb]
      d_vecs = [jnp.full((16,), d, jnp.int32) for d in range(_DIM)]

      @plsc.parallel_loop(0, _BLK, 16, unroll=4)
      def _(tl0):
        tl_vec = iota + tl0
        for d in range(_DIM):
          v = plsc.load_gather(r, [tl_vec, d_vecs[d]])
          otp[ob, d >> 3, d & 7, pl.ds(tl0, 16)] = v

    def fire_store(s, buf):
      pltpu.async_copy(otp.at[buf], out_hbm.at[s, :, jb], ssem.at[buf])

    def wait_store(s, buf):
      pltpu.make_async_copy(otp.at[buf], out_hbm.at[s, :, jb],
                            ssem.at[buf]).wait()

    for q in range(_NB - 1):
      extract(q, q)
      fire_gather(q)

    def step(i, carry):
      for q in range(_NB):
        s = _NB * i + q

        @pl.when(s + _NB - 1 < n_s)
        def _():
          extract(s + _NB - 1, (q + _NB - 1) % _NB)
          fire_gather((q + _NB - 1) % _NB)

        wait_gather(q)

        @pl.when(s >= 2)
        def _():
          wait_store(s - 2, q & 1)

        transpose(q, q & 1)
        fire_store(s, q & 1)
      return carry

    lax.fori_loop(0, n_s // _NB, step, 0)
    wait_store(n_s - 2, 0)
    wait_store(n_s - 1, 1)

  return pl.kernel(
      body,
      out_type=jax.ShapeDtypeStruct((n_s, _DIM // 8, _NW, 8, _BLK),
                                    jnp.float32),
      mesh=mesh,
      scratch_types=[
          pltpu.VMEM((n_tok,), jnp.int32),
          pltpu.VMEM((_NB, _BLK), jnp.int32),
          pltpu.VMEM((_NB, _BLK, _DIM), jnp.float32),
          pltpu.VMEM((2, _DIM // 8, 8, _BLK), jnp.float32),
          pltpu.SemaphoreType.DMA((_NB,)),
          pltpu.SemaphoreType.DMA((2,)),
      ],
      compiler_params=pltpu.CompilerParams(use_tc_tiling_on_sc=False,
                                           needs_layout_passes=False),
  )


def kernel(x, table):
  r, s = x.shape
  idx = x.reshape(-1).astype(jnp.int32)
  out5 = _make_lookup(s)(table, idx)
  return out5.transpose(2, 4, 0, 1, 3).reshape(r, s, _DIM)


# speedup vs baseline: 1.6521x; 1.6444x over previous
import jax
import jax.numpy as jnp
from jax import lax
from jax.experimental import pallas as pl
from jax.experimental.pallas import tpu as pltpu
from jax.experimental.pallas import tpu_sc as plsc

_DIM = 32
_TP = 129
_SP = 201
_NB = 4
_BLK = 128
_NC = 2
_NS = 16
_NW = _NC * _NS


def _make_lookup(n_s):
  mesh = plsc.VectorSubcoreMesh(
      core_axis_name="c", subcore_axis_name="s",
      num_cores=_NC, num_subcores=_NS)
  n_tok = n_s * _BLK

  def body(table_hbm, idx_hbm, out_hbm, idx_v, cidx, rows, otp, gsem, ssem):
    jb = lax.axis_index("s") * _NC + lax.axis_index("c")
    pltpu.sync_copy(idx_hbm.at[jb], idx_v.at[:, pl.ds(0, n_s)])

    iota = lax.iota(jnp.int32, 16)

    def extract(s, buf):
      s_vec = jnp.broadcast_to(s, (16,)).astype(jnp.int32)
      for g in range(_BLK // 16):
        tl_vec = iota + (g * 16)
        cidx[buf, pl.ds(g * 16, 16)] = plsc.load_gather(idx_v, [tl_vec, s_vec])

    def fire_gather(buf):
      pltpu.async_copy(table_hbm.at[cidx.at[buf]], rows.at[buf],
                       gsem.at[buf])

    def wait_gather(buf):
      pltpu.make_async_copy(table_hbm.at[pl.ds(0, _BLK), :], rows.at[buf],
                            gsem.at[buf]).wait()

    a_vecs = [(iota + 16 * h) >> 3 for h in range(2)]
    c_vecs = [(iota + 16 * h) & 7 for h in range(2)]

    def transpose(rb, ob):
      r = rows.at[rb]
      ob_vec = jnp.full((16,), ob, jnp.int32)

      @plsc.parallel_loop(0, _BLK, 1, unroll=8)
      def _(tl):
        tl_vec = jnp.broadcast_to(tl, (16,)).astype(jnp.int32)
        for h in range(2):
          v = r[tl, pl.ds(16 * h, 16)]
          plsc.store_scatter(otp, [ob_vec, a_vecs[h], c_vecs[h], tl_vec], v)

    def fire_store(s, buf):
      pltpu.async_copy(otp.at[buf, :, :, pl.ds(0, _BLK)],
                       out_hbm.at[s, :, jb], ssem.at[buf])

    def wait_store(s, buf):
      pltpu.make_async_copy(otp.at[buf, :, :, pl.ds(0, _BLK)],
                            out_hbm.at[s, :, jb], ssem.at[buf]).wait()

    for q in range(_NB - 1):
      extract(q, q)
      fire_gather(q)

    def step(i, carry):
      for q in range(_NB):
        s = _NB * i + q

        @pl.when(s + _NB - 1 < n_s)
        def _():
          extract(s + _NB - 1, (q + _NB - 1) % _NB)
          fire_gather((q + _NB - 1) % _NB)

        wait_gather(q)

        @pl.when(s >= 2)
        def _():
          wait_store(s - 2, q & 1)

        transpose(q, q & 1)
        fire_store(s, q & 1)
      return carry

    lax.fori_loop(0, n_s // _NB, step, 0)
    wait_store(n_s - 2, 0)
    wait_store(n_s - 1, 1)

  return pl.kernel(
      body,
      out_type=jax.ShapeDtypeStruct((n_s, _DIM // 8, _NW, 8, _BLK),
                                    jnp.float32),
      mesh=mesh,
      scratch_types=[
          pltpu.VMEM((_BLK, _SP), jnp.int32),
          pltpu.VMEM((_NB, _BLK), jnp.int32),
          pltpu.VMEM((_NB, _BLK, _DIM), jnp.float32),
          pltpu.VMEM((2, _DIM // 8, 8, _TP), jnp.float32),
          pltpu.SemaphoreType.DMA((_NB,)),
          pltpu.SemaphoreType.DMA((2,)),
      ],
      compiler_params=pltpu.CompilerParams(use_tc_tiling_on_sc=False,
                                           needs_layout_passes=False),
  )


def kernel(x, table):
  r, s = x.shape
  idx = x.reshape(_NW, _BLK, s).astype(jnp.int32)
  out5 = _make_lookup(s)(table, idx)
  return out5.transpose(2, 4, 0, 1, 3).reshape(r, s, _DIM)
